# baseline (speedup 1.0000x reference)
import functools

import jax
import jax.numpy as jnp
from jax import lax
from jax.experimental import pallas as pl
from jax.experimental.pallas import tpu as pltpu
from jax.experimental.pallas import tpu_sc as plsc

N = 10000
E = 320000
D = 128
NPAD = 10240
NCORE = 2
NSUB = 16
ROWS_PER_SUB = NPAD // NSUB
CHUNK = 64
CH_PER_W = 160
NBUF = 3
CH_HALF = CH_PER_W // 2
E2 = NCORE * NSUB * CH_PER_W * CHUNK
ZROWS = 16

BR = 1000
GRID = N // BR


def _segment_sum_sc(table, src, dst):
    mesh = plsc.VectorSubcoreMesh(core_axis_name="c", subcore_axis_name="s")

    @functools.partial(
        pl.kernel,
        out_type=[jax.ShapeDtypeStruct((NCORE, NPAD, D), jnp.float32),
                  jax.ShapeDtypeStruct((NCORE, NPAD), jnp.float32)],
        mesh=mesh,
        scratch_types=[
            pltpu.VMEM((2 * CH_HALF, 1, CHUNK), jnp.int32),
            pltpu.VMEM((NBUF, CHUNK, D), jnp.float32),
            pltpu.VMEM((CHUNK,), jnp.float32),
            pltpu.VMEM((ZROWS, D), jnp.float32),
            pltpu.VMEM((ROWS_PER_SUB,), jnp.float32),
            pltpu.VMEM_SHARED((NPAD, D), jnp.float32),
            pltpu.VMEM_SHARED((NPAD,), jnp.float32),
            pltpu.SemaphoreType.DMA((NBUF,)),
            pltpu.SemaphoreType.DMA,
        ],
    )
    def seg_kernel(tab_hbm, idx_hbm, agg_hbm, cnt_hbm,
                   idx_v, rows2, ones_v, zbuf, zcnt, acc_sh, cnt_sh,
                   sem2, sem_c):
        cid = lax.axis_index("c")
        sid = lax.axis_index("s")
        w = cid * NSUB + sid

        @pl.loop(0, ZROWS)
        def _(r):
            @pl.loop(0, D, step=16)
            def _(cc):
                zbuf[r, pl.ds(cc, 16)] = jnp.zeros((16,), jnp.float32)

        @pl.loop(0, ROWS_PER_SUB, step=16)
        def _(i):
            zcnt[pl.ds(i, 16)] = jnp.zeros((16,), jnp.float32)

        @pl.loop(0, CHUNK, step=16)
        def _(i):
            ones_v[pl.ds(i, 16)] = jnp.ones((16,), jnp.float32)

        pltpu.sync_copy(idx_hbm.at[pl.ds(w * 4 * CH_HALF, 2 * CH_HALF)], idx_v)
        for _pb in range(NBUF):
            pltpu.async_copy(tab_hbm.at[idx_v.at[2 * _pb, 0]], rows2.at[_pb],
                             sem2.at[_pb])

        @pl.loop(0, ROWS_PER_SUB // ZROWS)
        def _(j):
            pltpu.sync_copy(
                zbuf, acc_sh.at[pl.ds(sid * ROWS_PER_SUB + j * ZROWS, ZROWS)])

        pltpu.sync_copy(zcnt, cnt_sh.at[pl.ds(sid * ROWS_PER_SUB, ROWS_PER_SUB)])

        plsc.subcore_barrier()

        @pl.loop(0, 2)
        def _(h):
            @pl.when(h > 0)
            def _():
                pltpu.sync_copy(
                    idx_hbm.at[pl.ds(w * 4 * CH_HALF + h * 2 * CH_HALF,
                                     2 * CH_HALF)],
                    idx_v)
                for _pb in range(NBUF):
                    pltpu.async_copy(tab_hbm.at[idx_v.at[2 * _pb, 0]],
                                     rows2.at[_pb], sem2.at[_pb])

            @pl.loop(0, CH_HALF)
            def _(j):
                b = lax.rem(j, NBUF)
                jp = jnp.minimum(j + NBUF, CH_HALF - 1)
                pltpu.make_async_copy(tab_hbm.at[idx_v.at[2 * j, 0]],
                                      rows2.at[b], sem2.at[b]).wait()
                cdma = pltpu.async_copy(ones_v, cnt_sh.at[idx_v.at[2 * j + 1, 0]],
                                        sem_c, add=True)
                pltpu.sync_copy(rows2.at[b], acc_sh.at[idx_v.at[2 * j + 1, 0]],
                                add=True)
                pltpu.async_copy(tab_hbm.at[idx_v.at[2 * jp, 0]], rows2.at[b],
                                 sem2.at[b])
                cdma.wait()

            for _pb in range(NBUF):
                pltpu.make_async_copy(tab_hbm.at[idx_v.at[0, 0]], rows2.at[_pb],
                                      sem2.at[_pb]).wait()

        plsc.subcore_barrier()

        pltpu.sync_copy(acc_sh.at[pl.ds(sid * ROWS_PER_SUB, ROWS_PER_SUB)],
                        agg_hbm.at[cid, pl.ds(sid * ROWS_PER_SUB, ROWS_PER_SUB)])
        pltpu.sync_copy(cnt_sh.at[pl.ds(sid * ROWS_PER_SUB, ROWS_PER_SUB)],
                        cnt_hbm.at[cid, pl.ds(sid * ROWS_PER_SUB, ROWS_PER_SUB)])

    npadidx = jnp.arange(E2 - E, dtype=jnp.int32)
    src_p = jnp.concatenate([src, npadidx % N]).reshape(E2 // CHUNK, CHUNK)
    dst_p = jnp.concatenate([dst, N + npadidx % (NPAD - N)]
                            ).reshape(E2 // CHUNK, CHUNK)
    idx_p = jnp.stack([src_p, dst_p],
                      axis=1).reshape(2 * E2 // CHUNK, 1, CHUNK)
    return seg_kernel(table, idx_p)


def _matT(a, w):
    return lax.dot_general(a, w, (((1,), (1,)), ((), ())),
                           preferred_element_type=jnp.float32)


def _stage_a(x, W0l):
    def body(x_ref, w_ref, o_ref):
        o_ref[...] = _matT(x_ref[...], w_ref[...])

    return pl.pallas_call(
        body,
        grid=(GRID,),
        in_specs=[pl.BlockSpec((BR, D), lambda i: (i, 0)),
                  pl.BlockSpec((D, D), lambda i: (0, 0))],
        out_specs=pl.BlockSpec((BR, D), lambda i: (i, 0)),
        out_shape=jax.ShapeDtypeStruct((N, D), jnp.float32),
    )(x, W0l)


def _stage_b(agg, cnt, x, W0r, b0l, W1l):
    def body(agg_ref, cnt_ref, x_ref, w0r_ref, b0l_ref, w1l_ref, h_ref, hp_ref):
        a = agg_ref[0] + agg_ref[1]
        c = jnp.maximum(cnt_ref[0] + cnt_ref[1], 1.0)
        h = jnp.maximum(a / c + b0l_ref[...] + _matT(x_ref[...], w0r_ref[...]),
                        0.0)
        h_ref[...] = h
        hp_ref[...] = _matT(h, w1l_ref[...])

    return pl.pallas_call(
        body,
        grid=(GRID,),
        in_specs=[pl.BlockSpec((NCORE, BR, D), lambda i: (0, i, 0)),
                  pl.BlockSpec((NCORE, BR, 1), lambda i: (0, i, 0)),
                  pl.BlockSpec((BR, D), lambda i: (i, 0)),
                  pl.BlockSpec((D, D), lambda i: (0, 0)),
                  pl.BlockSpec((1, D), lambda i: (0, 0)),
                  pl.BlockSpec((D, D), lambda i: (0, 0))],
        out_specs=[pl.BlockSpec((BR, D), lambda i: (i, 0)),
                   pl.BlockSpec((BR, D), lambda i: (i, 0))],
        out_shape=[jax.ShapeDtypeStruct((N, D), jnp.float32),
                   jax.ShapeDtypeStruct((N, D), jnp.float32)],
    )(agg, cnt.reshape(NCORE, NPAD, 1), x, W0r, b0l.reshape(1, D), W1l)


def _stage_c(agg, cnt, h, W1r, b1l, Wp1, bp1, Wp2, bp2, Wc1, bc1, Wc2, bc2):
    def body(agg_ref, cnt_ref, h_ref, w1r_ref, b1l_ref, wp1_ref, bp1_ref,
             wp2_ref, bp2_ref, wc1_ref, bc1_ref, wc2_ref, bc2_ref, o_ref):
        a = agg_ref[0] + agg_ref[1]
        c = jnp.maximum(cnt_ref[0] + cnt_ref[1], 1.0)
        h2 = a / c + b1l_ref[...] + _matT(h_ref[...], w1r_ref[...])
        m = jnp.max(h2, axis=1, keepdims=True)
        lse = jnp.log(jnp.sum(jnp.exp(h2 - m), axis=1, keepdims=True)) + m
        e = h2 - lse
        p = jnp.maximum(_matT(e, wp1_ref[...]) + bp1_ref[...], 0.0)
        p = _matT(p, wp2_ref[...]) + bp2_ref[...]
        q = jnp.maximum(_matT(p, wc1_ref[...]) + bc1_ref[...], 0.0)
        o_ref[...] = (jnp.sum(q * wc2_ref[...], axis=1, keepdims=True)
                      + bc2_ref[...])

    full = lambda shape: pl.BlockSpec(shape, lambda i: tuple(0 for _ in shape))
    return pl.pallas_call(
        body,
        grid=(GRID,),
        in_specs=[pl.BlockSpec((NCORE, BR, D), lambda i: (0, i, 0)),
                  pl.BlockSpec((NCORE, BR, 1), lambda i: (0, i, 0)),
                  pl.BlockSpec((BR, D), lambda i: (i, 0)),
                  full((D, D)), full((1, D)),
                  full((D, D)), full((1, D)),
                  full((D, D)), full((1, D)),
                  full((32, D)), full((1, 32)),
                  full((1, 32)), full((1, 1))],
        out_specs=pl.BlockSpec((BR, 1), lambda i: (i, 0)),
        out_shape=jax.ShapeDtypeStruct((N, 1), jnp.float32),
    )(agg, cnt.reshape(NCORE, NPAD, 1), h, W1r, b1l.reshape(1, D),
      Wp1, bp1.reshape(1, D), Wp2, bp2.reshape(1, D), Wc1, bc1.reshape(1, 32),
      Wc2, bc2.reshape(1, 1))


def kernel(x, edge_index0, edge_index1, W0l, b0l, W0r, W1l, b1l, W1r,
           Wp1, bp1, Wp2, bp2, Wc1, bc1, Wc2, bc2):
    xp0 = _stage_a(x, W0l)
    agg0, cnt0 = _segment_sum_sc(xp0, edge_index0[0], edge_index0[1])
    h, hp1 = _stage_b(agg0, cnt0, x, W0r, b0l, W1l)
    agg1, cnt1 = _segment_sum_sc(hp1, edge_index1[0], edge_index1[1])
    return _stage_c(agg1, cnt1, h, W1r, b1l, Wp1, bp1, Wp2, bp2,
                    Wc1, bc1, Wc2, bc2)


# speedup vs baseline: 1.0284x; 1.0284x over previous
import functools

import jax
import jax.numpy as jnp
from jax import lax
from jax.experimental import pallas as pl
from jax.experimental.pallas import tpu as pltpu
from jax.experimental.pallas import tpu_sc as plsc

N = 10000
E = 320000
D = 128
NPAD = 10240
NCORE = 2
NSUB = 16
ROWS_PER_SUB = NPAD // NSUB
CHUNK = 128
CH_PER_W = 80
CH_HALF = CH_PER_W // 2
E2 = NCORE * NSUB * CH_PER_W * CHUNK
ZROWS = 32

BR = 1000
GRID = N // BR


def _segment_sum_sc(table, src, dst):
    mesh = plsc.VectorSubcoreMesh(core_axis_name="c", subcore_axis_name="s")

    @functools.partial(
        pl.kernel,
        out_type=[jax.ShapeDtypeStruct((NCORE, NPAD, D), jnp.float32),
                  jax.ShapeDtypeStruct((NCORE, NPAD), jnp.float32)],
        mesh=mesh,
        scratch_types=[
            pltpu.VMEM((2 * CH_HALF, 1, CHUNK), jnp.int32),
            pltpu.VMEM((2, CHUNK, D), jnp.float32),
            pltpu.VMEM((CHUNK,), jnp.float32),
            pltpu.VMEM((ZROWS, D), jnp.float32),
            pltpu.VMEM((ROWS_PER_SUB,), jnp.float32),
            pltpu.VMEM_SHARED((NPAD, D), jnp.float32),
            pltpu.VMEM_SHARED((NPAD,), jnp.float32),
            pltpu.SemaphoreType.DMA((2,)),
            pltpu.SemaphoreType.DMA,
        ],
    )
    def seg_kernel(tab_hbm, idx_hbm, agg_hbm, cnt_hbm,
                   idx_v, rows2, ones_v, zbuf, zcnt, acc_sh, cnt_sh,
                   sem2, sem_c):
        cid = lax.axis_index("c")
        sid = lax.axis_index("s")
        w = cid * NSUB + sid

        @pl.loop(0, ZROWS)
        def _(r):
            @pl.loop(0, D, step=16)
            def _(cc):
                zbuf[r, pl.ds(cc, 16)] = jnp.zeros((16,), jnp.float32)

        @pl.loop(0, ROWS_PER_SUB, step=16)
        def _(i):
            zcnt[pl.ds(i, 16)] = jnp.zeros((16,), jnp.float32)

        @pl.loop(0, CHUNK, step=16)
        def _(i):
            ones_v[pl.ds(i, 16)] = jnp.ones((16,), jnp.float32)

        pltpu.sync_copy(idx_hbm.at[pl.ds(w * 4 * CH_HALF, 2 * CH_HALF)], idx_v)
        pltpu.async_copy(tab_hbm.at[idx_v.at[0, 0]], rows2.at[0], sem2.at[0])
        pltpu.async_copy(tab_hbm.at[idx_v.at[2, 0]], rows2.at[1], sem2.at[1])

        @pl.loop(0, ROWS_PER_SUB // ZROWS)
        def _(j):
            pltpu.sync_copy(
                zbuf, acc_sh.at[pl.ds(sid * ROWS_PER_SUB + j * ZROWS, ZROWS)])

        pltpu.sync_copy(zcnt, cnt_sh.at[pl.ds(sid * ROWS_PER_SUB, ROWS_PER_SUB)])

        plsc.subcore_barrier()

        @pl.loop(0, 2)
        def _(h):
            @pl.when(h > 0)
            def _():
                pltpu.sync_copy(
                    idx_hbm.at[pl.ds(w * 4 * CH_HALF + h * 2 * CH_HALF,
                                     2 * CH_HALF)],
                    idx_v)
                pltpu.async_copy(tab_hbm.at[idx_v.at[0, 0]], rows2.at[0],
                                 sem2.at[0])
                pltpu.async_copy(tab_hbm.at[idx_v.at[2, 0]], rows2.at[1],
                                 sem2.at[1])

            @pl.loop(0, CH_HALF)
            def _(j):
                b = lax.rem(j, 2)
                jp = jnp.minimum(j + 2, CH_HALF - 1)
                pltpu.make_async_copy(tab_hbm.at[idx_v.at[2 * j, 0]],
                                      rows2.at[b], sem2.at[b]).wait()
                cdma = pltpu.async_copy(ones_v, cnt_sh.at[idx_v.at[2 * j + 1, 0]],
                                        sem_c, add=True)
                pltpu.sync_copy(rows2.at[b], acc_sh.at[idx_v.at[2 * j + 1, 0]],
                                add=True)
                pltpu.async_copy(tab_hbm.at[idx_v.at[2 * jp, 0]], rows2.at[b],
                                 sem2.at[b])
                cdma.wait()

            pltpu.make_async_copy(tab_hbm.at[idx_v.at[0, 0]], rows2.at[0],
                                  sem2.at[0]).wait()
            pltpu.make_async_copy(tab_hbm.at[idx_v.at[0, 0]], rows2.at[1],
                                  sem2.at[1]).wait()

        plsc.subcore_barrier()

        pltpu.sync_copy(acc_sh.at[pl.ds(sid * ROWS_PER_SUB, ROWS_PER_SUB)],
                        agg_hbm.at[cid, pl.ds(sid * ROWS_PER_SUB, ROWS_PER_SUB)])
        pltpu.sync_copy(cnt_sh.at[pl.ds(sid * ROWS_PER_SUB, ROWS_PER_SUB)],
                        cnt_hbm.at[cid, pl.ds(sid * ROWS_PER_SUB, ROWS_PER_SUB)])

    npadidx = jnp.arange(E2 - E, dtype=jnp.int32)
    src_p = jnp.concatenate([src, npadidx % N]).reshape(E2 // CHUNK, CHUNK)
    dst_p = jnp.concatenate([dst, N + npadidx % (NPAD - N)]
                            ).reshape(E2 // CHUNK, CHUNK)
    idx_p = jnp.stack([src_p, dst_p],
                      axis=1).reshape(2 * E2 // CHUNK, 1, CHUNK)
    return seg_kernel(table, idx_p)


def _matT(a, w):
    return lax.dot_general(a, w, (((1,), (1,)), ((), ())),
                           preferred_element_type=jnp.float32)


def _stage_a(x, W0l):
    def body(x_ref, w_ref, o_ref):
        o_ref[...] = _matT(x_ref[...], w_ref[...])

    return pl.pallas_call(
        body,
        grid=(GRID,),
        in_specs=[pl.BlockSpec((BR, D), lambda i: (i, 0)),
                  pl.BlockSpec((D, D), lambda i: (0, 0))],
        out_specs=pl.BlockSpec((BR, D), lambda i: (i, 0)),
        out_shape=jax.ShapeDtypeStruct((N, D), jnp.float32),
    )(x, W0l)


def _stage_b(agg, cnt, x, W0r, b0l, W1l):
    def body(agg_ref, cnt_ref, x_ref, w0r_ref, b0l_ref, w1l_ref, h_ref, hp_ref):
        a = agg_ref[0] + agg_ref[1]
        c = jnp.maximum(cnt_ref[0] + cnt_ref[1], 1.0)
        h = jnp.maximum(a / c + b0l_ref[...] + _matT(x_ref[...], w0r_ref[...]),
                        0.0)
        h_ref[...] = h
        hp_ref[...] = _matT(h, w1l_ref[...])

    return pl.pallas_call(
        body,
        grid=(GRID,),
        in_specs=[pl.BlockSpec((NCORE, BR, D), lambda i: (0, i, 0)),
                  pl.BlockSpec((NCORE, BR, 1), lambda i: (0, i, 0)),
                  pl.BlockSpec((BR, D), lambda i: (i, 0)),
                  pl.BlockSpec((D, D), lambda i: (0, 0)),
                  pl.BlockSpec((1, D), lambda i: (0, 0)),
                  pl.BlockSpec((D, D), lambda i: (0, 0))],
        out_specs=[pl.BlockSpec((BR, D), lambda i: (i, 0)),
                   pl.BlockSpec((BR, D), lambda i: (i, 0))],
        out_shape=[jax.ShapeDtypeStruct((N, D), jnp.float32),
                   jax.ShapeDtypeStruct((N, D), jnp.float32)],
    )(agg, cnt.reshape(NCORE, NPAD, 1), x, W0r, b0l.reshape(1, D), W1l)


def _stage_c(agg, cnt, h, W1r, b1l, Wp1, bp1, Wp2, bp2, Wc1, bc1, Wc2, bc2):
    def body(agg_ref, cnt_ref, h_ref, w1r_ref, b1l_ref, wp1_ref, bp1_ref,
             wp2_ref, bp2_ref, wc1_ref, bc1_ref, wc2_ref, bc2_ref, o_ref):
        a = agg_ref[0] + agg_ref[1]
        c = jnp.maximum(cnt_ref[0] + cnt_ref[1], 1.0)
        h2 = a / c + b1l_ref[...] + _matT(h_ref[...], w1r_ref[...])
        m = jnp.max(h2, axis=1, keepdims=True)
        lse = jnp.log(jnp.sum(jnp.exp(h2 - m), axis=1, keepdims=True)) + m
        e = h2 - lse
        p = jnp.maximum(_matT(e, wp1_ref[...]) + bp1_ref[...], 0.0)
        p = _matT(p, wp2_ref[...]) + bp2_ref[...]
        q = jnp.maximum(_matT(p, wc1_ref[...]) + bc1_ref[...], 0.0)
        o_ref[...] = (jnp.sum(q * wc2_ref[...], axis=1, keepdims=True)
                      + bc2_ref[...])

    full = lambda shape: pl.BlockSpec(shape, lambda i: tuple(0 for _ in shape))
    return pl.pallas_call(
        body,
        grid=(GRID,),
        in_specs=[pl.BlockSpec((NCORE, BR, D), lambda i: (0, i, 0)),
                  pl.BlockSpec((NCORE, BR, 1), lambda i: (0, i, 0)),
                  pl.BlockSpec((BR, D), lambda i: (i, 0)),
                  full((D, D)), full((1, D)),
                  full((D, D)), full((1, D)),
                  full((D, D)), full((1, D)),
                  full((32, D)), full((1, 32)),
                  full((1, 32)), full((1, 1))],
        out_specs=pl.BlockSpec((BR, 1), lambda i: (i, 0)),
        out_shape=jax.ShapeDtypeStruct((N, 1), jnp.float32),
    )(agg, cnt.reshape(NCORE, NPAD, 1), h, W1r, b1l.reshape(1, D),
      Wp1, bp1.reshape(1, D), Wp2, bp2.reshape(1, D), Wc1, bc1.reshape(1, 32),
      Wc2, bc2.reshape(1, 1))


def kernel(x, edge_index0, edge_index1, W0l, b0l, W0r, W1l, b1l, W1r,
           Wp1, bp1, Wp2, bp2, Wc1, bc1, Wc2, bc2):
    xp0 = _stage_a(x, W0l)
    agg0, cnt0 = _segment_sum_sc(xp0, edge_index0[0], edge_index0[1])
    h, hp1 = _stage_b(agg0, cnt0, x, W0r, b0l, W1l)
    agg1, cnt1 = _segment_sum_sc(hp1, edge_index1[0], edge_index1[1])
    return _stage_c(agg1, cnt1, h, W1r, b1l, Wp1, bp1, Wp2, bp2,
                    Wc1, bc1, Wc2, bc2)
